# baseline (speedup 1.0000x reference)
import jax
import jax.numpy as jnp
from jax import lax
from jax.experimental import pallas as pl
from jax.experimental.pallas import tpu as pltpu
from jax.experimental.pallas import tpu_sc as plsc


BATCH = 4
SEQ_LEN = 8192
DIM = 768
ROWS = BATCH * SEQ_LEN
NUM_CORES = 2
NUM_SUBCORES = 16
NW = NUM_CORES * NUM_SUBCORES
ROWS_PER_W = ROWS // NW
CHUNK = 32
N_CHUNKS = ROWS_PER_W // CHUNK
W_PER_BATCH = SEQ_LEN // ROWS_PER_W


T_PER_W = SEQ_LEN // NW
TCH = 16
NCH = T_PER_W // TCH
STAGES = NCH * BATCH
NXB = 8
PREF = 4
_PROBE_NO_ADD = False


def _sc_body(x_hbm, pos_hbm, out_hbm, xbufs, posbufs, semp, semx, semo):
    cid = lax.axis_index("c")
    sid = lax.axis_index("s")
    wid = sid * NUM_CORES + cid
    t_base = wid * T_PER_W

    def start_pos(c, slot):
        pltpu.async_copy(
            pos_hbm.at[pl.ds(t_base + c * TCH, TCH)], posbufs[slot],
            semp[slot])

    def wait_pos(slot):
        pltpu.make_async_copy(
            pos_hbm.at[pl.ds(0, TCH)], posbufs[slot], semp[slot]).wait()

    def start_x(c, b, slot):
        r0 = b * SEQ_LEN + t_base + c * TCH
        pltpu.async_copy(x_hbm.at[pl.ds(r0, TCH)], xbufs[slot], semx[slot])

    def wait_x(slot):
        pltpu.make_async_copy(
            x_hbm.at[pl.ds(0, TCH)], xbufs[slot], semx[slot]).wait()

    def start_out(c, b, slot):
        r0 = b * SEQ_LEN + t_base + c * TCH
        pltpu.async_copy(xbufs[slot], out_hbm.at[pl.ds(r0, TCH)], semo[slot])

    def wait_out(slot):
        pltpu.make_async_copy(
            xbufs[slot], out_hbm.at[pl.ds(0, TCH)], semo[slot]).wait()

    def add_stage(slot, pslot):
        xb, pb = xbufs[slot], posbufs[pslot]

        def row_add(r, acc):
            for j in range(DIM // 16):
                plsc.addupdate(
                    xb.at[r, pl.ds(j * 16, 16)], pb[r, pl.ds(j * 16, 16)])
            return acc
        if _PROBE_NO_ADD:
            return
        lax.fori_loop(0, TCH, row_add, 0)

    def stage_body(oct_, u, first_octet=False, last_octet=False):
        c = 2 * oct_ + u // 4
        b = u % 4
        pslot = (u // 4) % 2
        if b == 0:
            wait_pos(pslot)
        if b == 1 and not (last_octet and u == 5):
            start_pos(c + 1, (pslot + 1) % 2)
        wait_x(u % NXB)
        add_stage(u % NXB, pslot)
        start_out(c, b, u % NXB)
        if not (last_octet and u >= 8 - PREF):
            if not (first_octet and u < PREF):
                wait_out((u + PREF) % NXB)
            off = u + PREF
            start_x(2 * oct_ + off // 4, off % 4, off % NXB)

    n_oct = STAGES // 8
    start_pos(0, 0)
    for u in range(PREF):
        start_x(u // 4, u % 4, u % NXB)
    for u in range(8):
        stage_body(0, u, first_octet=True)

    def octet(q, carry):
        for u in range(8):
            stage_body(q, u)
        return carry

    lax.fori_loop(1, n_oct - 1, octet, 0)
    for u in range(8):
        stage_body(n_oct - 1, u, last_octet=True)
    for slot in range(NXB):
        wait_out(slot)


def _sc_kernel(x, position_embed):
    x2 = x.reshape(ROWS, DIM)
    mesh = plsc.VectorSubcoreMesh(
        core_axis_name="c", subcore_axis_name="s",
        num_cores=NUM_CORES, num_subcores=NUM_SUBCORES,
    )
    out = pl.kernel(
        _sc_body,
        out_type=jax.ShapeDtypeStruct((ROWS, DIM), jnp.float32),
        mesh=mesh,
        scratch_types=[
            [pltpu.VMEM((TCH, DIM), jnp.float32) for _ in range(NXB)],
            [pltpu.VMEM((TCH, DIM), jnp.float32) for _ in range(2)],
            [pltpu.SemaphoreType.DMA for _ in range(2)],
            [pltpu.SemaphoreType.DMA for _ in range(NXB)],
            [pltpu.SemaphoreType.DMA for _ in range(NXB)],
        ],
    )(x2, position_embed)
    return out.reshape(BATCH, SEQ_LEN, DIM)


SC_T = 1024
SC_TPW = SC_T // NW
SC_NCH = SC_TPW // TCH
SC_STAGES = SC_NCH * BATCH
SC_NXB = 4
SC_PREF = 2


def _sc_partial_body(x_hbm, pos_hbm, out_hbm, xbufs, posbufs, semp, semx, semo):
    cid = lax.axis_index("c")
    sid = lax.axis_index("s")
    wid = sid * NUM_CORES + cid
    t_base = wid * SC_TPW

    def start_pos(c):
        pltpu.async_copy(
            pos_hbm.at[pl.ds(t_base + c * TCH, TCH)], posbufs[c], semp[c])

    def wait_pos(c):
        pltpu.make_async_copy(
            pos_hbm.at[pl.ds(0, TCH)], posbufs[c], semp[c]).wait()

    def start_x(c, b, slot):
        r0 = b * SEQ_LEN + t_base + c * TCH
        pltpu.async_copy(x_hbm.at[pl.ds(r0, TCH)], xbufs[slot], semx[slot])

    def wait_x(slot):
        pltpu.make_async_copy(
            x_hbm.at[pl.ds(0, TCH)], xbufs[slot], semx[slot]).wait()

    def start_out(c, b, slot):
        r0 = b * SEQ_LEN + t_base + c * TCH
        pltpu.async_copy(xbufs[slot], out_hbm.at[pl.ds(r0, TCH)], semo[slot])

    def wait_out(slot):
        pltpu.make_async_copy(
            xbufs[slot], out_hbm.at[pl.ds(0, TCH)], semo[slot]).wait()

    def add_stage(slot, pslot):
        xb, pb = xbufs[slot], posbufs[pslot]

        def row_add(r, acc):
            for j in range(DIM // 16):
                plsc.addupdate(
                    xb.at[r, pl.ds(j * 16, 16)], pb[r, pl.ds(j * 16, 16)])
            return acc
        lax.fori_loop(0, TCH, row_add, 0)

    start_pos(0)
    start_pos(1)
    for s in range(SC_PREF):
        start_x(s // BATCH, s % BATCH, s % SC_NXB)
    for s in range(SC_STAGES):
        c, b = divmod(s, BATCH)
        if b == 0:
            wait_pos(c)
        wait_x(s % SC_NXB)
        add_stage(s % SC_NXB, c)
        start_out(c, b, s % SC_NXB)
        if s + SC_PREF < SC_STAGES:
            if s + SC_PREF - SC_NXB >= 0:
                wait_out((s + SC_PREF) % SC_NXB)
            s2 = s + SC_PREF
            start_x(s2 // BATCH, s2 % BATCH, s2 % SC_NXB)
    for slot in range(SC_NXB):
        wait_out(slot)


def _sc_partial(x2, position_embed):
    mesh = plsc.VectorSubcoreMesh(
        core_axis_name="c", subcore_axis_name="s",
        num_cores=NUM_CORES, num_subcores=NUM_SUBCORES,
    )
    return pl.kernel(
        _sc_partial_body,
        out_type=jax.ShapeDtypeStruct((ROWS, DIM), jnp.float32),
        mesh=mesh,
        scratch_types=[
            [pltpu.VMEM((TCH, DIM), jnp.float32) for _ in range(SC_NXB)],
            [pltpu.VMEM((TCH, DIM), jnp.float32) for _ in range(2)],
            [pltpu.SemaphoreType.DMA for _ in range(2)],
            [pltpu.SemaphoreType.DMA for _ in range(SC_NXB)],
            [pltpu.SemaphoreType.DMA for _ in range(SC_NXB)],
        ],
    )(x2, position_embed)


_TC_SB = 512
_TC_OFF = SC_T // _TC_SB


def _tc_finish_body(x_ref, pos_ref, done_ref, out_ref):
    del done_ref
    out_ref[...] = x_ref[...] + pos_ref[...][None, :, :]


def _tc_finish(x, position_embed, sc_out):
    n_i = (SEQ_LEN - SC_T) // _TC_SB
    return pl.pallas_call(
        _tc_finish_body,
        grid=(n_i, BATCH),
        in_specs=[
            pl.BlockSpec((1, _TC_SB, DIM), lambda i, b: (b, i + _TC_OFF, 0)),
            pl.BlockSpec((_TC_SB, DIM), lambda i, b: (i + _TC_OFF, 0)),
            pl.BlockSpec(memory_space=pl.ANY),
        ],
        out_specs=pl.BlockSpec((1, _TC_SB, DIM), lambda i, b: (b, i + _TC_OFF, 0)),
        out_shape=jax.ShapeDtypeStruct((BATCH, SEQ_LEN, DIM), jnp.float32),
        input_output_aliases={2: 0},
        compiler_params=pltpu.CompilerParams(
            dimension_semantics=("arbitrary", "arbitrary"),
        ),
    )(x, position_embed, sc_out)


def _hybrid_kernel(x, position_embed):
    x2 = x.reshape(ROWS, DIM)
    sc_out = _sc_partial(x2, position_embed).reshape(BATCH, SEQ_LEN, DIM)
    return _tc_finish(x, position_embed, sc_out)


_SEQ_BLOCK = 512


def _tc_add_body(x_ref, pos_ref, out_ref):
    out_ref[...] = x_ref[...] + pos_ref[...][None, :, :]


def _tc_kernel(x, position_embed):
    batch, seq_len, dim = x.shape
    grid = (seq_len // _SEQ_BLOCK,)
    return pl.pallas_call(
        _tc_add_body,
        grid=grid,
        in_specs=[
            pl.BlockSpec((batch, _SEQ_BLOCK, dim), lambda i: (0, i, 0)),
            pl.BlockSpec((_SEQ_BLOCK, dim), lambda i: (i, 0)),
        ],
        out_specs=pl.BlockSpec((batch, _SEQ_BLOCK, dim), lambda i: (0, i, 0)),
        out_shape=jax.ShapeDtypeStruct(x.shape, x.dtype),
        compiler_params=pltpu.CompilerParams(
            dimension_semantics=("parallel",),
        ),
    )(x, position_embed)


def kernel(x, position_embed):
    return _hybrid_kernel(x, position_embed)


# speedup vs baseline: 1.0537x; 1.0537x over previous
import jax
import jax.numpy as jnp
from jax import lax
from jax.experimental import pallas as pl
from jax.experimental.pallas import tpu as pltpu
from jax.experimental.pallas import tpu_sc as plsc


BATCH = 4
SEQ_LEN = 8192
DIM = 768
ROWS = BATCH * SEQ_LEN
NUM_CORES = 2
NUM_SUBCORES = 16
NW = NUM_CORES * NUM_SUBCORES

T_PER_W = SEQ_LEN // NW
TCH = 16
NCH = T_PER_W // TCH
STAGES = NCH * BATCH
NXB = 8
PREF = 4


def _sc_body(x_hbm, pos_hbm, out_hbm, xbufs, posbufs, semp, semx, semo):
    cid = lax.axis_index("c")
    sid = lax.axis_index("s")
    wid = sid * NUM_CORES + cid
    t_base = wid * T_PER_W

    def start_pos(c, slot):
        pltpu.async_copy(
            pos_hbm.at[pl.ds(t_base + c * TCH, TCH)], posbufs[slot],
            semp[slot])

    def wait_pos(slot):
        pltpu.make_async_copy(
            pos_hbm.at[pl.ds(0, TCH)], posbufs[slot], semp[slot]).wait()

    def start_x(c, b, slot):
        r0 = b * SEQ_LEN + t_base + c * TCH
        pltpu.async_copy(x_hbm.at[pl.ds(r0, TCH)], xbufs[slot], semx[slot])

    def wait_x(slot):
        pltpu.make_async_copy(
            x_hbm.at[pl.ds(0, TCH)], xbufs[slot], semx[slot]).wait()

    def start_out(c, b, slot):
        r0 = b * SEQ_LEN + t_base + c * TCH
        pltpu.async_copy(xbufs[slot], out_hbm.at[pl.ds(r0, TCH)], semo[slot])

    def wait_out(slot):
        pltpu.make_async_copy(
            xbufs[slot], out_hbm.at[pl.ds(0, TCH)], semo[slot]).wait()

    def add_stage(slot, pslot):
        xb, pb = xbufs[slot], posbufs[pslot]

        def row_add(r, acc):
            for j in range(DIM // 16):
                plsc.addupdate(
                    xb.at[r, pl.ds(j * 16, 16)], pb[r, pl.ds(j * 16, 16)])
            return acc
        lax.fori_loop(0, TCH, row_add, 0)

    def stage_body(oct_, u, first_octet=False, last_octet=False):
        c = 2 * oct_ + u // 4
        b = u % 4
        pslot = (u // 4) % 2
        if b == 0:
            wait_pos(pslot)
        if b == 1 and not (last_octet and u == 5):
            start_pos(c + 1, (pslot + 1) % 2)
        wait_x(u % NXB)
        add_stage(u % NXB, pslot)
        start_out(c, b, u % NXB)
        if not (last_octet and u >= 8 - PREF):
            if not (first_octet and u < PREF):
                wait_out((u + PREF) % NXB)
            off = u + PREF
            start_x(2 * oct_ + off // 4, off % 4, off % NXB)

    n_oct = STAGES // 8
    start_pos(0, 0)
    for u in range(PREF):
        start_x(u // 4, u % 4, u % NXB)
    for u in range(8):
        stage_body(0, u, first_octet=True)

    def octet(q, carry):
        for u in range(8):
            stage_body(q, u)
        return carry

    lax.fori_loop(1, n_oct - 1, octet, 0)
    for u in range(8):
        stage_body(n_oct - 1, u, last_octet=True)
    for slot in range(NXB):
        wait_out(slot)


def kernel(x, position_embed):
    x2 = x.reshape(ROWS, DIM)
    mesh = plsc.VectorSubcoreMesh(
        core_axis_name="c", subcore_axis_name="s",
        num_cores=NUM_CORES, num_subcores=NUM_SUBCORES,
    )
    out = pl.kernel(
        _sc_body,
        out_type=jax.ShapeDtypeStruct((ROWS, DIM), jnp.float32),
        mesh=mesh,
        scratch_types=[
            [pltpu.VMEM((TCH, DIM), jnp.float32) for _ in range(NXB)],
            [pltpu.VMEM((TCH, DIM), jnp.float32) for _ in range(2)],
            [pltpu.SemaphoreType.DMA for _ in range(2)],
            [pltpu.SemaphoreType.DMA for _ in range(NXB)],
            [pltpu.SemaphoreType.DMA for _ in range(NXB)],
        ],
    )(x2, position_embed)
    return out.reshape(BATCH, SEQ_LEN, DIM)
